# baseline (speedup 1.0000x reference)
import functools

import jax
import jax.numpy as jnp
from jax import lax
from jax.experimental import pallas as pl
from jax.experimental.pallas import tpu as pltpu
from jax.experimental.pallas import tpu_sc as plsc

BATCH = 4096
HIST = 50
EMBED_DIM = 64

NC = 2
NS = 16
NW = NC * NS

B_TOTAL = BATCH * HIST
B_PER_W = B_TOTAL // NW
G = 128
NCH = B_PER_W // G
L_ROWS = B_TOTAL // 2
L_PER_CH = G // 2


def _gather_body(x_hbm, table_hbm, l_hbm, idx_v, rows0, rows1, sem0, sem1):
    cid = lax.axis_index("c")
    sid = lax.axis_index("s")
    wid = sid * NC + cid
    lbase = wid * (B_PER_W // 2)

    pltpu.sync_copy(x_hbm.at[wid], idx_v)

    def fire(j, rows, sem):
        pltpu.async_copy(table_hbm.at[idx_v.at[j, 0]], rows.at[:L_PER_CH], sem)
        pltpu.async_copy(table_hbm.at[idx_v.at[j, 1]], rows.at[L_PER_CH:], sem)

    def drain(j, rows, sem):
        pltpu.make_async_copy(
            table_hbm.at[idx_v.at[j, 0]], rows.at[:L_PER_CH], sem
        ).wait()
        pltpu.make_async_copy(
            table_hbm.at[idx_v.at[j, 1]], rows.at[L_PER_CH:], sem
        ).wait()

    def store(j, rows):
        dst = l_hbm.at[pl.ds(lbase + j * L_PER_CH, L_PER_CH)]
        pltpu.sync_copy(rows.at[:L_PER_CH], dst.at[:, :EMBED_DIM])
        pltpu.sync_copy(rows.at[L_PER_CH:], dst.at[:, EMBED_DIM:])

    fire(0, rows0, sem0)

    @pl.loop(0, NCH, step=2)
    def _(j):
        drain(j, rows0, sem0)
        fire(j + 1, rows1, sem1)
        store(j, rows0)
        drain(j + 1, rows1, sem1)

        @pl.when(j + 2 < NCH)
        def _():
            fire(j + 2, rows0, sem0)

        store(j + 1, rows1)


@jax.jit
def _sc_gather(x_flat, table):
    mesh = plsc.VectorSubcoreMesh(core_axis_name="c", subcore_axis_name="s")
    f = functools.partial(
        pl.kernel,
        out_type=jax.ShapeDtypeStruct((L_ROWS, 2 * EMBED_DIM), jnp.float32),
        mesh=mesh,
        scratch_types=[
            pltpu.VMEM((NCH, 2, L_PER_CH), jnp.int32),
            pltpu.VMEM((G, EMBED_DIM), jnp.float32),
            pltpu.VMEM((G, EMBED_DIM), jnp.float32),
            pltpu.SemaphoreType.DMA,
            pltpu.SemaphoreType.DMA,
        ],
        compiler_params=pltpu.CompilerParams(use_tc_tiling_on_sc=False),
    )(_gather_body)
    x_split = x_flat.reshape(NW, NCH, L_PER_CH, 2).transpose(0, 1, 3, 2)
    return f(x_split, table)


TC_BB = 128
TC_LB = TC_BB * HIST // 2


def _finish_body(l_ref, x_ref, o_ref, m_ref):
    e = l_ref[...]
    a = e[:, :EMBED_DIM].reshape(TC_BB, HIST // 2, 1, EMBED_DIM)
    b = e[:, EMBED_DIM:].reshape(TC_BB, HIST // 2, 1, EMBED_DIM)
    o_ref[...] = jnp.concatenate([a, b], axis=2).reshape(TC_BB, HIST, EMBED_DIM)
    m_ref[...] = x_ref[...] >= 1


@jax.jit
def _tc_finish(l, x):
    return pl.pallas_call(
        _finish_body,
        grid=(BATCH // TC_BB,),
        in_specs=[
            pl.BlockSpec((TC_LB, 2 * EMBED_DIM), lambda i: (i, 0)),
            pl.BlockSpec((TC_BB, HIST), lambda i: (i, 0)),
        ],
        out_specs=[
            pl.BlockSpec((TC_BB, HIST, EMBED_DIM), lambda i: (i, 0, 0)),
            pl.BlockSpec((TC_BB, HIST), lambda i: (i, 0)),
        ],
        out_shape=[
            jax.ShapeDtypeStruct((BATCH, HIST, EMBED_DIM), jnp.float32),
            jax.ShapeDtypeStruct((BATCH, HIST), jnp.bool_),
        ],
    )(l, x)


def kernel(x, weight_matrix):
    l = _sc_gather(x, weight_matrix)
    embeds, mask = _tc_finish(l, x)
    return embeds, mask
